# baseline (speedup 1.0000x reference)
import functools

import jax
import jax.numpy as jnp
from jax import lax
from jax.experimental import pallas as pl
from jax.experimental.pallas import tpu as pltpu
from jax.experimental.pallas import tpu_sc as plsc

NUM_UNITS = 16
NEURONS = 256
TIMESTEPS = 64
HIDDEN = 128
DIRECT_WEIGHT = 1.5
FLAT = TIMESTEPS * NEURONS
K_TOTAL = NUM_UNITS * FLAT

SC_ROWS = 8 * FLAT
TC_CHUNK = 4096
TC_CHUNKS = (K_TOTAL - SC_ROWS) // TC_CHUNK

SC_CORES = 2
SC_SUBCORES = 16
SC_WORKERS = SC_CORES * SC_SUBCORES
RPW = SC_ROWS // SC_WORKERS
SC_CH = 256
SC_NCH = RPW // SC_CH


def _unit_fwd_kernel(spikes_ref, unitw_ref, uo_ref, injT_ref):
    u = pl.program_id(0)

    @pl.when(u < NUM_UNITS)
    def _inject():
        inj = jnp.dot(spikes_ref[:], unitw_ref[0],
                      preferred_element_type=jnp.float32)
        injT_ref[:, pl.ds(u, 1), :] = inj.reshape(TIMESTEPS, 1, NEURONS)

    @pl.when(u == NUM_UNITS)
    def _scan():
        def step(t, mem):
            m = mem * 0.9 + injT_ref[pl.ds(t, 1), :, :].reshape(
                NUM_UNITS, NEURONS)
            spk = jax.nn.sigmoid(4.0 * (m - 1.0))
            uo_ref[:, pl.ds(t * NEURONS, NEURONS)] = spk
            return m - spk

        lax.fori_loop(0, TIMESTEPS, step,
                      jnp.zeros((NUM_UNITS, NEURONS), jnp.float32))


def _unit_forward(input_spikes, unit_W):
    return pl.pallas_call(
        _unit_fwd_kernel,
        grid=(NUM_UNITS + 1,),
        in_specs=[
            pl.BlockSpec((TIMESTEPS, NEURONS), lambda u: (0, 0)),
            pl.BlockSpec((1, NEURONS, NEURONS),
                         lambda u: (jnp.minimum(u, NUM_UNITS - 1), 0, 0)),
        ],
        out_specs=pl.BlockSpec((NUM_UNITS, FLAT), lambda u: (0, 0)),
        out_shape=jax.ShapeDtypeStruct((NUM_UNITS, FLAT), jnp.float32),
        scratch_shapes=[
            pltpu.VMEM((TIMESTEPS, NUM_UNITS, NEURONS), jnp.float32),
        ],
    )(input_spikes, unit_W)


def _sc_gemv_body(v_hbm, w1_hbm, out_hbm, v_vmem, wb0, wb1, acc_vmem,
                  sem0, sem1):
    c = lax.axis_index("c")
    s = lax.axis_index("s")
    wid = s * SC_CORES + c
    base = wid * RPW
    pltpu.sync_copy(v_hbm.at[pl.ds(base, RPW)], v_vmem)

    bufs = (wb0, wb1)
    sems = (sem0, sem1)
    for b in range(2):
        pltpu.async_copy(
            w1_hbm.at[pl.ds((base + b * SC_CH) * HIDDEN, SC_CH * HIDDEN)],
            bufs[b], sems[b])
    groups = SC_CH // 16

    def pair_step(p, accs):
        for b in range(2):
            ci = 2 * p + b
            pltpu.make_async_copy(
                w1_hbm.at[pl.ds(0, SC_CH * HIDDEN)], bufs[b], sems[b]).wait()
            buf = bufs[b]

            def group_step(g, a):
                v16 = v_vmem[pl.ds(ci * SC_CH + g * 16, 16)]
                a0, a1 = a
                for l in range(8):
                    sv = v16[l]
                    rowoff = (g * 16 + l) * HIDDEN
                    a0 = tuple(a0[j] + sv * buf[pl.ds(rowoff + j * 16, 16)]
                               for j in range(8))
                for l in range(8, 16):
                    sv = v16[l]
                    rowoff = (g * 16 + l) * HIDDEN
                    a1 = tuple(a1[j] + sv * buf[pl.ds(rowoff + j * 16, 16)]
                               for j in range(8))
                return (a0, a1)

            accs = plsc.parallel_loop(0, groups, carry=accs,
                                      unroll=2)(group_step)

            @pl.when(ci + 2 < SC_NCH)
            def _start_next():
                pltpu.async_copy(
                    w1_hbm.at[pl.ds((base + (ci + 2) * SC_CH) * HIDDEN,
                                    SC_CH * HIDDEN)],
                    buf, sems[b])
        return accs

    zero8 = tuple(jnp.zeros((16,), jnp.float32) for _ in range(8))
    a0, a1 = lax.fori_loop(0, SC_NCH // 2, pair_step, (zero8, zero8))
    for j in range(8):
        acc_vmem[pl.ds(j * 16, 16)] = a0[j] + a1[j]
    pltpu.sync_copy(acc_vmem, out_hbm.at[wid])


def _sc_gemv(v_flat, coord_W1):
    mesh = plsc.VectorSubcoreMesh(core_axis_name="c", subcore_axis_name="s",
                                  num_cores=SC_CORES,
                                  num_subcores=SC_SUBCORES)
    fn = functools.partial(
        pl.kernel, mesh=mesh,
        out_type=jax.ShapeDtypeStruct((SC_WORKERS, HIDDEN), jnp.float32),
        scratch_types=[
            pltpu.VMEM((RPW,), jnp.float32),
            pltpu.VMEM((SC_CH * HIDDEN,), jnp.float32),
            pltpu.VMEM((SC_CH * HIDDEN,), jnp.float32),
            pltpu.VMEM((HIDDEN,), jnp.float32),
            pltpu.SemaphoreType.DMA,
            pltpu.SemaphoreType.DMA,
        ],
    )(_sc_gemv_body)
    return fn(v_flat, coord_W1.reshape(-1))


def _tc_gemv_kernel(uo_ref, w1_ref, out_ref):
    g = pl.program_id(0)

    @pl.when(g == 0)
    def _init():
        out_ref[:] = jnp.zeros_like(out_ref)

    k0 = SC_ROWS + g * TC_CHUNK
    u = k0 // FLAT
    off = k0 - u * FLAT
    v = uo_ref[pl.ds(u, 1), pl.ds(off, TC_CHUNK)]
    out_ref[:] += jnp.dot(v, w1_ref[0],
                          preferred_element_type=jnp.float32)


def _tc_gemv(uo, coord_W1):
    w1_blocked = coord_W1.reshape(K_TOTAL // TC_CHUNK, TC_CHUNK, HIDDEN)
    sc_blk = SC_ROWS // TC_CHUNK
    return pl.pallas_call(
        _tc_gemv_kernel,
        grid=(TC_CHUNKS,),
        in_specs=[
            pl.BlockSpec((NUM_UNITS, FLAT), lambda g: (0, 0)),
            pl.BlockSpec((1, TC_CHUNK, HIDDEN),
                         lambda g: (sc_blk + g, 0, 0)),
        ],
        out_specs=pl.BlockSpec((1, HIDDEN), lambda g: (0, 0)),
        out_shape=jax.ShapeDtypeStruct((1, HIDDEN), jnp.float32),
    )(uo, w1_blocked)


def _head_kernel(uo_ref, tc_acc_ref, partials_ref, spikes_ref, b1_ref,
                 w2_ref, b2_ref, u42_ref, u7_ref, u9_ref, out_ref):
    h_pre = tc_acc_ref[:] + jnp.sum(partials_ref[:], axis=0,
                                    keepdims=True) + b1_ref[:]
    h = jnp.tanh(h_pre)
    logits = jnp.dot(h, w2_ref[:],
                     preferred_element_type=jnp.float32) + b2_ref[:]
    probs = jax.nn.sigmoid(logits)
    sample = (u42_ref[:] < probs).astype(jnp.float32)
    k_idx = lax.broadcasted_iota(jnp.int32,
                                 (NUM_UNITS * NUM_UNITS, NUM_UNITS), 0)
    j_idx = lax.broadcasted_iota(jnp.int32,
                                 (NUM_UNITS * NUM_UNITS, NUM_UNITS), 1)
    sel = (lax.rem(k_idx, NUM_UNITS) == j_idx).astype(jnp.float32)
    coeff = 3.0 * jnp.dot(sample, sel,
                          preferred_element_type=jnp.float32)
    final = jnp.dot(coeff, uo_ref[:],
                    preferred_element_type=jnp.float32)
    sm = jnp.mean(spikes_ref[:])
    p = jnp.clip(sm + 0.02, 0.0, 1.0)
    tgt = (u7_ref[:] < p).astype(jnp.float32)
    final = final * 0.5 + tgt * DIRECT_WEIGHT
    mean_f = jnp.mean(final)
    target_mean = sm * 10.0 + 0.2
    boost = jnp.where(mean_f < 0.2,
                      jnp.maximum(0.0, target_mean - mean_f), 0.0)
    out_ref[:] = final + u9_ref[:] * boost * 2.0


def _head(uo, tc_acc, partials, input_spikes, b1, W2, b2, u42, u7, u9):
    full = lambda g: (0, 0)
    return pl.pallas_call(
        _head_kernel,
        grid=(1,),
        in_specs=[
            pl.BlockSpec((NUM_UNITS, FLAT), full),
            pl.BlockSpec((1, HIDDEN), full),
            pl.BlockSpec((SC_WORKERS, HIDDEN), full),
            pl.BlockSpec((TIMESTEPS, NEURONS), full),
            pl.BlockSpec((1, HIDDEN), full),
            pl.BlockSpec((HIDDEN, NUM_UNITS * NUM_UNITS), full),
            pl.BlockSpec((1, NUM_UNITS * NUM_UNITS), full),
            pl.BlockSpec((1, NUM_UNITS * NUM_UNITS), full),
            pl.BlockSpec((1, FLAT), full),
            pl.BlockSpec((1, FLAT), full),
        ],
        out_specs=pl.BlockSpec((1, FLAT), full),
        out_shape=jax.ShapeDtypeStruct((1, FLAT), jnp.float32),
    )(uo, tc_acc, partials, input_spikes, b1, W2, b2, u42, u7, u9)


def _const_uniform(seed, shape):
    import numpy as np
    return np.asarray(jax.random.uniform(jax.random.key(seed), shape))


_U42 = _const_uniform(42, (NUM_UNITS, NUM_UNITS)).reshape(1, -1)
_U7 = _const_uniform(7, (TIMESTEPS, NEURONS)).reshape(1, -1)
_U9 = _const_uniform(9, (TIMESTEPS, NEURONS)).reshape(1, -1)


@jax.jit
def _run(input_spikes, unit_W, coord_W1, coord_b1, coord_W2, coord_b2):
    u42 = _U42
    u7 = _U7
    u9 = _U9
    uo = _unit_forward(input_spikes, unit_W)
    partials = _sc_gemv(uo.reshape(K_TOTAL), coord_W1)
    tc_acc = _tc_gemv(uo, coord_W1)
    out = _head(uo, tc_acc, partials, input_spikes,
                coord_b1.reshape(1, HIDDEN), coord_W2,
                coord_b2.reshape(1, -1), u42, u7, u9)
    return out.reshape(TIMESTEPS, NEURONS)


def kernel(input_spikes, unit_W, coord_W1, coord_b1, coord_W2, coord_b2):
    return _run(input_spikes, unit_W, coord_W1, coord_b1, coord_W2, coord_b2)


# speedup vs baseline: 1.0193x; 1.0193x over previous
import functools

import jax
import jax.numpy as jnp
from jax import lax
from jax.experimental import pallas as pl
from jax.experimental.pallas import tpu as pltpu
from jax.experimental.pallas import tpu_sc as plsc

NUM_UNITS = 16
NEURONS = 256
TIMESTEPS = 64
HIDDEN = 128
DIRECT_WEIGHT = 1.5
FLAT = TIMESTEPS * NEURONS
K_TOTAL = NUM_UNITS * FLAT

SC_ROWS = 8 * FLAT
TC_CHUNK = 4096
TC_CHUNKS = (K_TOTAL - SC_ROWS) // TC_CHUNK

SC_CORES = 2
SC_SUBCORES = 16
SC_WORKERS = SC_CORES * SC_SUBCORES
RPW = SC_ROWS // SC_WORKERS
SC_CH = 256
SC_NCH = RPW // SC_CH


def _unit_fwd_kernel(spikes_ref, unitw_ref, uo_ref, injT_ref):
    u = pl.program_id(0)

    @pl.when(u < NUM_UNITS)
    def _inject():
        inj = jnp.dot(spikes_ref[:], unitw_ref[0],
                      preferred_element_type=jnp.float32)
        injT_ref[:, pl.ds(u, 1), :] = inj.reshape(TIMESTEPS, 1, NEURONS)

    @pl.when(u == NUM_UNITS)
    def _scan():
        def step(t, mem):
            m = mem * 0.9 + injT_ref[pl.ds(t, 1), :, :].reshape(
                NUM_UNITS, NEURONS)
            spk = jax.nn.sigmoid(4.0 * (m - 1.0))
            uo_ref[:, pl.ds(t * NEURONS, NEURONS)] = spk
            return m - spk

        lax.fori_loop(0, TIMESTEPS, step,
                      jnp.zeros((NUM_UNITS, NEURONS), jnp.float32))


def _unit_forward(input_spikes, unit_W):
    return pl.pallas_call(
        _unit_fwd_kernel,
        grid=(NUM_UNITS + 1,),
        in_specs=[
            pl.BlockSpec((TIMESTEPS, NEURONS), lambda u: (0, 0)),
            pl.BlockSpec((1, NEURONS, NEURONS),
                         lambda u: (jnp.minimum(u, NUM_UNITS - 1), 0, 0)),
        ],
        out_specs=pl.BlockSpec((NUM_UNITS, FLAT), lambda u: (0, 0)),
        out_shape=jax.ShapeDtypeStruct((NUM_UNITS, FLAT), jnp.float32),
        scratch_shapes=[
            pltpu.VMEM((TIMESTEPS, NUM_UNITS, NEURONS), jnp.float32),
        ],
    )(input_spikes, unit_W)


def _sc_gemv_body(v_hbm, w1_hbm, out_hbm, v_vmem, wb0, wb1, acc_vmem,
                  sem0, sem1):
    c = lax.axis_index("c")
    s = lax.axis_index("s")
    wid = s * SC_CORES + c
    base = wid * RPW
    pltpu.sync_copy(v_hbm.at[pl.ds(base, RPW)], v_vmem)

    bufs = (wb0, wb1)
    sems = (sem0, sem1)
    for b in range(2):
        pltpu.async_copy(
            w1_hbm.at[pl.ds((base + b * SC_CH) * HIDDEN, SC_CH * HIDDEN)],
            bufs[b], sems[b])
    groups = SC_CH // 16

    def pair_step(p, accs):
        for b in range(2):
            ci = 2 * p + b
            pltpu.make_async_copy(
                w1_hbm.at[pl.ds(0, SC_CH * HIDDEN)], bufs[b], sems[b]).wait()
            buf = bufs[b]

            def group_step(g, a):
                v16 = v_vmem[pl.ds(ci * SC_CH + g * 16, 16)]
                a0, a1 = a
                for l in range(8):
                    sv = v16[l]
                    rowoff = (g * 16 + l) * HIDDEN
                    a0 = tuple(a0[j] + sv * buf[pl.ds(rowoff + j * 16, 16)]
                               for j in range(8))
                for l in range(8, 16):
                    sv = v16[l]
                    rowoff = (g * 16 + l) * HIDDEN
                    a1 = tuple(a1[j] + sv * buf[pl.ds(rowoff + j * 16, 16)]
                               for j in range(8))
                return (a0, a1)

            accs = plsc.parallel_loop(0, groups, carry=accs,
                                      unroll=2)(group_step)

            @pl.when(ci + 2 < SC_NCH)
            def _start_next():
                pltpu.async_copy(
                    w1_hbm.at[pl.ds((base + (ci + 2) * SC_CH) * HIDDEN,
                                    SC_CH * HIDDEN)],
                    buf, sems[b])
        return accs

    zero8 = tuple(jnp.zeros((16,), jnp.float32) for _ in range(8))
    a0, a1 = lax.fori_loop(0, SC_NCH // 2, pair_step, (zero8, zero8))
    for j in range(8):
        acc_vmem[pl.ds(j * 16, 16)] = a0[j] + a1[j]
    pltpu.sync_copy(acc_vmem, out_hbm.at[wid])


def _sc_gemv(v_flat, coord_W1):
    mesh = plsc.VectorSubcoreMesh(core_axis_name="c", subcore_axis_name="s",
                                  num_cores=SC_CORES,
                                  num_subcores=SC_SUBCORES)
    fn = functools.partial(
        pl.kernel, mesh=mesh,
        out_type=jax.ShapeDtypeStruct((SC_WORKERS, HIDDEN), jnp.float32),
        scratch_types=[
            pltpu.VMEM((RPW,), jnp.float32),
            pltpu.VMEM((SC_CH * HIDDEN,), jnp.float32),
            pltpu.VMEM((SC_CH * HIDDEN,), jnp.float32),
            pltpu.VMEM((HIDDEN,), jnp.float32),
            pltpu.SemaphoreType.DMA,
            pltpu.SemaphoreType.DMA,
        ],
    )(_sc_gemv_body)
    return fn(v_flat, coord_W1.reshape(-1))


def _tc_gemv_kernel(uo_ref, w1_ref, out_ref):
    g = pl.program_id(0)

    @pl.when(g == 0)
    def _init():
        out_ref[:] = jnp.zeros_like(out_ref)

    k0 = SC_ROWS + g * TC_CHUNK
    u = k0 // FLAT
    off = k0 - u * FLAT
    v = uo_ref[pl.ds(u, 1), pl.ds(off, TC_CHUNK)]
    out_ref[:] += jnp.dot(v, w1_ref[0],
                          preferred_element_type=jnp.float32)


def _tc_gemv(uo, coord_W1):
    w1_blocked = coord_W1.reshape(K_TOTAL // TC_CHUNK, TC_CHUNK, HIDDEN)
    sc_blk = SC_ROWS // TC_CHUNK
    return pl.pallas_call(
        _tc_gemv_kernel,
        grid=(TC_CHUNKS,),
        in_specs=[
            pl.BlockSpec((NUM_UNITS, FLAT), lambda g: (0, 0)),
            pl.BlockSpec((1, TC_CHUNK, HIDDEN),
                         lambda g: (sc_blk + g, 0, 0)),
        ],
        out_specs=pl.BlockSpec((1, HIDDEN), lambda g: (0, 0)),
        out_shape=jax.ShapeDtypeStruct((1, HIDDEN), jnp.float32),
    )(uo, w1_blocked)


def _head_kernel(uo_ref, tc_acc_ref, partials_ref, spikes_ref, b1_ref,
                 w2_ref, b2_ref, u42_ref, u7_ref, u9_ref, out_ref):
    h_pre = tc_acc_ref[:] + jnp.sum(partials_ref[:], axis=0,
                                    keepdims=True) + b1_ref[:]
    h = jnp.tanh(h_pre)
    logits = jnp.dot(h, w2_ref[:],
                     preferred_element_type=jnp.float32) + b2_ref[:]
    probs = jax.nn.sigmoid(logits)
    sample = (u42_ref[:] < probs).astype(jnp.float32)
    k_idx = lax.broadcasted_iota(jnp.int32,
                                 (NUM_UNITS * NUM_UNITS, NUM_UNITS), 0)
    j_idx = lax.broadcasted_iota(jnp.int32,
                                 (NUM_UNITS * NUM_UNITS, NUM_UNITS), 1)
    sel = (lax.rem(k_idx, NUM_UNITS) == j_idx).astype(jnp.float32)
    coeff = 3.0 * jnp.dot(sample, sel,
                          preferred_element_type=jnp.float32)
    final = jnp.dot(coeff, uo_ref[:],
                    preferred_element_type=jnp.float32)
    sm = jnp.mean(spikes_ref[:])
    p = jnp.clip(sm + 0.02, 0.0, 1.0)
    tgt = (u7_ref[:] < p).astype(jnp.float32)
    final = final * 0.5 + tgt * DIRECT_WEIGHT
    mean_f = jnp.mean(final)
    target_mean = sm * 10.0 + 0.2
    boost = jnp.where(mean_f < 0.2,
                      jnp.maximum(0.0, target_mean - mean_f), 0.0)
    out_ref[:] = final + u9_ref[:] * boost * 2.0


def _head(uo, tc_acc, partials, input_spikes, b1, W2, b2, u42, u7, u9):
    full = lambda g: (0, 0)
    return pl.pallas_call(
        _head_kernel,
        grid=(1,),
        in_specs=[
            pl.BlockSpec((NUM_UNITS, FLAT), full),
            pl.BlockSpec((1, HIDDEN), full),
            pl.BlockSpec((SC_WORKERS, HIDDEN), full),
            pl.BlockSpec((TIMESTEPS, NEURONS), full),
            pl.BlockSpec((1, HIDDEN), full),
            pl.BlockSpec((HIDDEN, NUM_UNITS * NUM_UNITS), full),
            pl.BlockSpec((1, NUM_UNITS * NUM_UNITS), full),
            pl.BlockSpec((1, NUM_UNITS * NUM_UNITS), full),
            pl.BlockSpec((1, FLAT), full),
            pl.BlockSpec((1, FLAT), full),
        ],
        out_specs=pl.BlockSpec((1, FLAT), full),
        out_shape=jax.ShapeDtypeStruct((1, FLAT), jnp.float32),
    )(uo, tc_acc, partials, input_spikes, b1, W2, b2, u42, u7, u9)


def _threefry2x32(k0, k1, x0, x1):
    import numpy as np
    x0 = x0.astype(np.uint32).copy()
    x1 = x1.astype(np.uint32).copy()
    ks = [np.uint32(k0), np.uint32(k1),
          np.uint32(np.uint32(k0) ^ np.uint32(k1) ^ np.uint32(0x1BD11BDA))]
    rotations = [(13, 15, 26, 6), (17, 29, 16, 24)]
    x0 = (x0 + ks[0]).astype(np.uint32)
    x1 = (x1 + ks[1]).astype(np.uint32)
    for i in range(5):
        for r in rotations[i % 2]:
            x0 = (x0 + x1).astype(np.uint32)
            x1 = ((x1 << np.uint32(r)) | (x1 >> np.uint32(32 - r))).astype(
                np.uint32) ^ x0
        x0 = (x0 + ks[(i + 1) % 3]).astype(np.uint32)
        x1 = (x1 + ks[(i + 2) % 3] + np.uint32(i + 1)).astype(np.uint32)
    return x0, x1


def _const_uniform(seed, shape):
    import numpy as np
    n = int(np.prod(shape))
    b0, b1 = _threefry2x32(0, np.uint32(seed), np.zeros(n, np.uint32),
                           np.arange(n, dtype=np.uint32))
    bits = b0 ^ b1
    fl = ((bits >> np.uint32(9)) | np.uint32(0x3F800000)).view(np.float32)
    return np.maximum(0.0, fl - 1.0).reshape(shape).astype(np.float32)


_U42 = _const_uniform(42, (NUM_UNITS, NUM_UNITS)).reshape(1, -1)
_U7 = _const_uniform(7, (TIMESTEPS, NEURONS)).reshape(1, -1)
_U9 = _const_uniform(9, (TIMESTEPS, NEURONS)).reshape(1, -1)


@jax.jit
def _run(input_spikes, unit_W, coord_W1, coord_b1, coord_W2, coord_b2):
    u42 = _U42
    u7 = _U7
    u9 = _U9
    uo = _unit_forward(input_spikes, unit_W)
    partials = _sc_gemv(uo.reshape(K_TOTAL), coord_W1)
    tc_acc = _tc_gemv(uo, coord_W1)
    out = _head(uo, tc_acc, partials, input_spikes,
                coord_b1.reshape(1, HIDDEN), coord_W2,
                coord_b2.reshape(1, -1), u42, u7, u9)
    return out.reshape(TIMESTEPS, NEURONS)


def kernel(input_spikes, unit_W, coord_W1, coord_b1, coord_W2, coord_b2):
    return _run(input_spikes, unit_W, coord_W1, coord_b1, coord_W2, coord_b2)


# speedup vs baseline: 1.1083x; 1.0873x over previous
import functools

import jax
import jax.numpy as jnp
from jax import lax
from jax.experimental import pallas as pl
from jax.experimental.pallas import tpu as pltpu
from jax.experimental.pallas import tpu_sc as plsc

NUM_UNITS = 16
NEURONS = 256
TIMESTEPS = 64
HIDDEN = 128
DIRECT_WEIGHT = 1.5
FLAT = TIMESTEPS * NEURONS
K_TOTAL = NUM_UNITS * FLAT

SC_ROWS = 8 * FLAT
TC_CHUNK = 4096
TC_CHUNKS = (K_TOTAL - SC_ROWS) // TC_CHUNK

SC_CORES = 2
SC_SUBCORES = 16
SC_WORKERS = SC_CORES * SC_SUBCORES
RPW = SC_ROWS // SC_WORKERS
SC_CH = 256
SC_NCH = RPW // SC_CH


def _unit_fwd_kernel(spikes_ref, unitw_ref, uo_ref, injT_ref):
    x = spikes_ref[:]
    for u in range(NUM_UNITS):
        inj = jnp.dot(x, unitw_ref[u],
                      preferred_element_type=jnp.float32)
        injT_ref[:, pl.ds(u, 1), :] = inj.reshape(TIMESTEPS, 1, NEURONS)

    def step(t, mem):
        m = mem * 0.9 + injT_ref[pl.ds(t, 1), :, :].reshape(
            NUM_UNITS, NEURONS)
        spk = jax.nn.sigmoid(4.0 * (m - 1.0))
        uo_ref[:, pl.ds(t * NEURONS, NEURONS)] = spk
        return m - spk

    lax.fori_loop(0, TIMESTEPS, step,
                  jnp.zeros((NUM_UNITS, NEURONS), jnp.float32))


def _unit_forward(input_spikes, unit_W):
    return pl.pallas_call(
        _unit_fwd_kernel,
        grid=(1,),
        in_specs=[
            pl.BlockSpec((TIMESTEPS, NEURONS), lambda g: (0, 0)),
            pl.BlockSpec((NUM_UNITS, NEURONS, NEURONS),
                         lambda g: (0, 0, 0)),
        ],
        out_specs=pl.BlockSpec((NUM_UNITS, FLAT), lambda g: (0, 0)),
        out_shape=jax.ShapeDtypeStruct((NUM_UNITS, FLAT), jnp.float32),
        scratch_shapes=[
            pltpu.VMEM((TIMESTEPS, NUM_UNITS, NEURONS), jnp.float32),
        ],
    )(input_spikes, unit_W)


def _sc_gemv_body(v_hbm, w1_hbm, out_hbm, v_vmem, wb0, wb1, acc_vmem,
                  sem0, sem1):
    c = lax.axis_index("c")
    s = lax.axis_index("s")
    wid = s * SC_CORES + c
    base = wid * RPW
    pltpu.sync_copy(v_hbm.at[pl.ds(base, RPW)], v_vmem)

    bufs = (wb0, wb1)
    sems = (sem0, sem1)
    for b in range(2):
        pltpu.async_copy(
            w1_hbm.at[pl.ds((base + b * SC_CH) * HIDDEN, SC_CH * HIDDEN)],
            bufs[b], sems[b])
    groups = SC_CH // 16

    def pair_step(p, accs):
        for b in range(2):
            ci = 2 * p + b
            pltpu.make_async_copy(
                w1_hbm.at[pl.ds(0, SC_CH * HIDDEN)], bufs[b], sems[b]).wait()
            buf = bufs[b]

            def group_step(g, a):
                v16 = v_vmem[pl.ds(ci * SC_CH + g * 16, 16)]
                a0, a1 = a
                for l in range(8):
                    sv = v16[l]
                    rowoff = (g * 16 + l) * HIDDEN
                    a0 = tuple(a0[j] + sv * buf[pl.ds(rowoff + j * 16, 16)]
                               for j in range(8))
                for l in range(8, 16):
                    sv = v16[l]
                    rowoff = (g * 16 + l) * HIDDEN
                    a1 = tuple(a1[j] + sv * buf[pl.ds(rowoff + j * 16, 16)]
                               for j in range(8))
                return (a0, a1)

            accs = plsc.parallel_loop(0, groups, carry=accs,
                                      unroll=2)(group_step)

            @pl.when(ci + 2 < SC_NCH)
            def _start_next():
                pltpu.async_copy(
                    w1_hbm.at[pl.ds((base + (ci + 2) * SC_CH) * HIDDEN,
                                    SC_CH * HIDDEN)],
                    buf, sems[b])
        return accs

    zero8 = tuple(jnp.zeros((16,), jnp.float32) for _ in range(8))
    a0, a1 = lax.fori_loop(0, SC_NCH // 2, pair_step, (zero8, zero8))
    for j in range(8):
        acc_vmem[pl.ds(j * 16, 16)] = a0[j] + a1[j]
    pltpu.sync_copy(acc_vmem, out_hbm.at[wid])


def _sc_gemv(v_flat, coord_W1):
    mesh = plsc.VectorSubcoreMesh(core_axis_name="c", subcore_axis_name="s",
                                  num_cores=SC_CORES,
                                  num_subcores=SC_SUBCORES)
    fn = functools.partial(
        pl.kernel, mesh=mesh,
        out_type=jax.ShapeDtypeStruct((SC_WORKERS, HIDDEN), jnp.float32),
        scratch_types=[
            pltpu.VMEM((RPW,), jnp.float32),
            pltpu.VMEM((SC_CH * HIDDEN,), jnp.float32),
            pltpu.VMEM((SC_CH * HIDDEN,), jnp.float32),
            pltpu.VMEM((HIDDEN,), jnp.float32),
            pltpu.SemaphoreType.DMA,
            pltpu.SemaphoreType.DMA,
        ],
    )(_sc_gemv_body)
    return fn(v_flat, coord_W1.reshape(-1))


def _tc_gemv_kernel(uo_ref, w1_ref, out_ref):
    g = pl.program_id(0)

    @pl.when(g == 0)
    def _init():
        out_ref[:] = jnp.zeros_like(out_ref)

    k0 = SC_ROWS + g * TC_CHUNK
    u = k0 // FLAT
    off = k0 - u * FLAT
    v = uo_ref[pl.ds(u, 1), pl.ds(off, TC_CHUNK)]
    out_ref[:] += jnp.dot(v, w1_ref[0],
                          preferred_element_type=jnp.float32)


def _tc_gemv(uo, coord_W1):
    w1_blocked = coord_W1.reshape(K_TOTAL // TC_CHUNK, TC_CHUNK, HIDDEN)
    sc_blk = SC_ROWS // TC_CHUNK
    return pl.pallas_call(
        _tc_gemv_kernel,
        grid=(TC_CHUNKS,),
        in_specs=[
            pl.BlockSpec((NUM_UNITS, FLAT), lambda g: (0, 0)),
            pl.BlockSpec((1, TC_CHUNK, HIDDEN),
                         lambda g: (sc_blk + g, 0, 0)),
        ],
        out_specs=pl.BlockSpec((1, HIDDEN), lambda g: (0, 0)),
        out_shape=jax.ShapeDtypeStruct((1, HIDDEN), jnp.float32),
    )(uo, w1_blocked)


def _head_kernel(uo_ref, tc_acc_ref, partials_ref, spikes_ref, b1_ref,
                 w2_ref, b2_ref, u42_ref, u7_ref, u9_ref, out_ref):
    h_pre = tc_acc_ref[:] + jnp.sum(partials_ref[:], axis=0,
                                    keepdims=True) + b1_ref[:]
    h = jnp.tanh(h_pre)
    logits = jnp.dot(h, w2_ref[:],
                     preferred_element_type=jnp.float32) + b2_ref[:]
    probs = jax.nn.sigmoid(logits)
    sample = (u42_ref[:] < probs).astype(jnp.float32)
    k_idx = lax.broadcasted_iota(jnp.int32,
                                 (NUM_UNITS * NUM_UNITS, NUM_UNITS), 0)
    j_idx = lax.broadcasted_iota(jnp.int32,
                                 (NUM_UNITS * NUM_UNITS, NUM_UNITS), 1)
    sel = (lax.rem(k_idx, NUM_UNITS) == j_idx).astype(jnp.float32)
    coeff = 3.0 * jnp.dot(sample, sel,
                          preferred_element_type=jnp.float32)
    final = jnp.dot(coeff, uo_ref[:],
                    preferred_element_type=jnp.float32)
    sm = jnp.mean(spikes_ref[:])
    p = jnp.clip(sm + 0.02, 0.0, 1.0)
    tgt = (u7_ref[:] < p).astype(jnp.float32)
    final = final * 0.5 + tgt * DIRECT_WEIGHT
    mean_f = jnp.mean(final)
    target_mean = sm * 10.0 + 0.2
    boost = jnp.where(mean_f < 0.2,
                      jnp.maximum(0.0, target_mean - mean_f), 0.0)
    out_ref[:] = final + u9_ref[:] * boost * 2.0


def _head(uo, tc_acc, partials, input_spikes, b1, W2, b2, u42, u7, u9):
    full = lambda g: (0, 0)
    return pl.pallas_call(
        _head_kernel,
        grid=(1,),
        in_specs=[
            pl.BlockSpec((NUM_UNITS, FLAT), full),
            pl.BlockSpec((1, HIDDEN), full),
            pl.BlockSpec((SC_WORKERS, HIDDEN), full),
            pl.BlockSpec((TIMESTEPS, NEURONS), full),
            pl.BlockSpec((1, HIDDEN), full),
            pl.BlockSpec((HIDDEN, NUM_UNITS * NUM_UNITS), full),
            pl.BlockSpec((1, NUM_UNITS * NUM_UNITS), full),
            pl.BlockSpec((1, NUM_UNITS * NUM_UNITS), full),
            pl.BlockSpec((1, FLAT), full),
            pl.BlockSpec((1, FLAT), full),
        ],
        out_specs=pl.BlockSpec((1, FLAT), full),
        out_shape=jax.ShapeDtypeStruct((1, FLAT), jnp.float32),
    )(uo, tc_acc, partials, input_spikes, b1, W2, b2, u42, u7, u9)


def _threefry2x32(k0, k1, x0, x1):
    import numpy as np
    x0 = x0.astype(np.uint32).copy()
    x1 = x1.astype(np.uint32).copy()
    ks = [np.uint32(k0), np.uint32(k1),
          np.uint32(np.uint32(k0) ^ np.uint32(k1) ^ np.uint32(0x1BD11BDA))]
    rotations = [(13, 15, 26, 6), (17, 29, 16, 24)]
    x0 = (x0 + ks[0]).astype(np.uint32)
    x1 = (x1 + ks[1]).astype(np.uint32)
    for i in range(5):
        for r in rotations[i % 2]:
            x0 = (x0 + x1).astype(np.uint32)
            x1 = ((x1 << np.uint32(r)) | (x1 >> np.uint32(32 - r))).astype(
                np.uint32) ^ x0
        x0 = (x0 + ks[(i + 1) % 3]).astype(np.uint32)
        x1 = (x1 + ks[(i + 2) % 3] + np.uint32(i + 1)).astype(np.uint32)
    return x0, x1


def _const_uniform(seed, shape):
    import numpy as np
    n = int(np.prod(shape))
    b0, b1 = _threefry2x32(0, np.uint32(seed), np.zeros(n, np.uint32),
                           np.arange(n, dtype=np.uint32))
    bits = b0 ^ b1
    fl = ((bits >> np.uint32(9)) | np.uint32(0x3F800000)).view(np.float32)
    return np.maximum(0.0, fl - 1.0).reshape(shape).astype(np.float32)


_U42 = _const_uniform(42, (NUM_UNITS, NUM_UNITS)).reshape(1, -1)
_U7 = _const_uniform(7, (TIMESTEPS, NEURONS)).reshape(1, -1)
_U9 = _const_uniform(9, (TIMESTEPS, NEURONS)).reshape(1, -1)


@jax.jit
def _run(input_spikes, unit_W, coord_W1, coord_b1, coord_W2, coord_b2):
    u42 = _U42
    u7 = _U7
    u9 = _U9
    uo = _unit_forward(input_spikes, unit_W)
    partials = _sc_gemv(uo.reshape(K_TOTAL), coord_W1)
    tc_acc = _tc_gemv(uo, coord_W1)
    out = _head(uo, tc_acc, partials, input_spikes,
                coord_b1.reshape(1, HIDDEN), coord_W2,
                coord_b2.reshape(1, -1), u42, u7, u9)
    return out.reshape(TIMESTEPS, NEURONS)


def kernel(input_spikes, unit_W, coord_W1, coord_b1, coord_W2, coord_b2):
    return _run(input_spikes, unit_W, coord_W1, coord_b1, coord_W2, coord_b2)
